# Initial kernel scaffold; baseline (speedup 1.0000x reference)
import jax
import jax.numpy as jnp
from jax.experimental import pallas as pl


def kernel(item_embeddings, price_embeddings, category_embeddings, samples, sampleLen, Wk_item, Wv_item, Wk_price, Wv_price):
    raise NotImplementedError("write your pallas kernel here")


# speedup vs baseline: 2.8708x; 2.8708x over previous
import functools
import math

import jax
import jax.numpy as jnp
from jax import lax
from jax.experimental import pallas as pl
from jax.experimental.pallas import tpu as pltpu, tpu_sc as plsc

NC, NS, L = 2, 16, 16
NB, BK, D = 16384, 20, 32
NOCC = NB * BK
N_IT, N_PR = 1_000_000, 100_000

_MESH = plsc.VectorSubcoreMesh(core_axis_name="c", subcore_axis_name="s",
                               num_cores=NC, num_subcores=NS)
_CP = pltpu.CompilerParams(use_tc_tiling_on_sc=False, needs_layout_passes=False)


_OPT = NOCC // (NC * NS)
_CHA = 2048


@functools.partial(
    pl.kernel, mesh=_MESH, compiler_params=_CP,
    out_type=(
        jax.ShapeDtypeStruct((NOCC, D), jnp.float32),
        jax.ShapeDtypeStruct((NOCC, D), jnp.float32),
        jax.ShapeDtypeStruct((NOCC,), jnp.int32),
    ),
    scratch_types=[
        pltpu.VMEM((_CHA,), jnp.int32),
        pltpu.VMEM((_CHA,), jnp.int32),
        pltpu.VMEM((_CHA, D), jnp.float32),
        pltpu.SemaphoreType.DMA,
    ],
)
def _phase_a(sf_hbm, item_hbm, price_hbm, qi_hbm, qp_hbm, pid_hbm,
             ids_v, pids_v, rows_v, sem):
  cid = lax.axis_index("c")
  sid = lax.axis_index("s")
  wid = cid * NS + sid

  def chunk(c, _):
    off = wid * _OPT + c * _CHA
    pltpu.sync_copy(sf_hbm.at[pl.ds(off, _CHA)], ids_v)
    pltpu.async_copy(item_hbm.at[ids_v], rows_v, sem).wait()
    pltpu.sync_copy(rows_v, qi_hbm.at[pl.ds(off, _CHA)])

    def pv(j, _):
      iv = ids_v[pl.ds(j * L, L)]
      pids_v[pl.ds(j * L, L)] = iv % N_PR
      return 0
    lax.fori_loop(0, _CHA // L, pv, 0)
    pltpu.sync_copy(pids_v, pid_hbm.at[pl.ds(off, _CHA)])
    pltpu.async_copy(price_hbm.at[pids_v], rows_v, sem).wait()
    pltpu.sync_copy(rows_v, qp_hbm.at[pl.ds(off, _CHA)])
    return 0

  lax.fori_loop(0, _OPT // _CHA, chunk, 0)


_BB = 512
_ISQ = 1.0 / math.sqrt(D)


def _phase_b_body(qi_ref, qp_ref, wki_ref, wvi_ref, wkp_ref, wvp_ref,
                  bi_ref, ei_ref, evi_ref, bp_ref, ep_ref, evp_ref):
  def tab(qref, wkref, wvref, bref, eref, evref):
    q = qref[...]
    s = jnp.sum(q, axis=1) * (1.0 / BK)
    b = jnp.tanh(s)
    k = jnp.dot(b, wkref[...], preferred_element_type=jnp.float32)
    v = jnp.dot(b, wvref[...], preferred_element_type=jnp.float32)
    sc = jnp.sum(q * k[:, None, :], axis=-1) * _ISQ
    e = jnp.exp(sc)
    bref[...] = b
    eref[...] = e
    evref[...] = e[:, :, None] * v[:, None, :]

  tab(qi_ref, wki_ref, wvi_ref, bi_ref, ei_ref, evi_ref)
  tab(qp_ref, wkp_ref, wvp_ref, bp_ref, ep_ref, evp_ref)


def _phase_b(qi3, qp3, wki, wvi, wkp, wvp):
  wspec = pl.BlockSpec((D, D), lambda i: (0, 0))
  return pl.pallas_call(
      _phase_b_body,
      grid=(NB // _BB,),
      in_specs=[
          pl.BlockSpec((_BB, BK, D), lambda i: (i, 0, 0)),
          pl.BlockSpec((_BB, BK, D), lambda i: (i, 0, 0)),
          wspec, wspec, wspec, wspec,
      ],
      out_specs=[
          pl.BlockSpec((_BB, D), lambda i: (i, 0)),
          pl.BlockSpec((_BB, BK), lambda i: (i, 0)),
          pl.BlockSpec((_BB, BK, D), lambda i: (i, 0, 0)),
          pl.BlockSpec((_BB, D), lambda i: (i, 0)),
          pl.BlockSpec((_BB, BK), lambda i: (i, 0)),
          pl.BlockSpec((_BB, BK, D), lambda i: (i, 0, 0)),
      ],
      out_shape=[
          jax.ShapeDtypeStruct((NB, D), jnp.float32),
          jax.ShapeDtypeStruct((NB, BK), jnp.float32),
          jax.ShapeDtypeStruct((NB, BK, D), jnp.float32),
          jax.ShapeDtypeStruct((NB, D), jnp.float32),
          jax.ShapeDtypeStruct((NB, BK), jnp.float32),
          jax.ShapeDtypeStruct((NB, BK, D), jnp.float32),
      ],
  )(qi3, qp3, wki, wvi, wkp, wvp)


_SOCC = NOCC // NS
_CH = 1024
_NCHUNK = _SOCC // _CH
_PB = 256
_FCH = 400
_Q = 4096
_HST = _Q + _CH + 16
_WCAP = _Q + _PB + 16
_WTRASH = _Q + _PB
_G1CAP = _CH + _PB + 16
_G1TRASH = _CH + _PB
_FCAP = _FCH + 16 + 16
_FTRASH = _FCH + 16
_SPAD_I, _SPAD_P = 524_288, 65_536


def _compress(m, cursor, trash, pairs):
  mi = m.astype(jnp.int32)
  c = plsc.cumsum(mi)
  n = jnp.sum(mi)
  pos = jnp.where(m, cursor + c - 1, trash)
  for ref, vec in pairs:
    plsc.store_scatter(ref, [pos], vec)
  return cursor + n


def _make_phase_c(n_rows, cw, windows, spad, linear_fin):
  half = n_rows // 2
  span = windows * cw
  fw = cw // NS

  scratch = [
      pltpu.VMEM((_CH,), jnp.int32),
      pltpu.VMEM((_CH,), jnp.float32),
      pltpu.VMEM((_CH,), jnp.int32),
      pltpu.VMEM((_CH,), jnp.float32),
      pltpu.VMEM((_CH,), jnp.float32),
      pltpu.VMEM((_CH,), jnp.float32),
      pltpu.VMEM((_G1CAP,), jnp.int32),
      pltpu.VMEM((_G1CAP,), jnp.int32),
      pltpu.VMEM((_HST,), jnp.int32),
      pltpu.VMEM((_HST,), jnp.int32),
      pltpu.VMEM((_WCAP,), jnp.int32),
      pltpu.VMEM((_WCAP,), jnp.int32),
      pltpu.VMEM((_PB, D), jnp.float32),
      pltpu.VMEM((_PB,), jnp.int32),
      pltpu.VMEM((16,), jnp.int32),
      pltpu.VMEM((16,), jnp.int32),
      pltpu.VMEM((_FCH + 16,), jnp.float32),
      pltpu.VMEM((_FCH + 16,), jnp.float32),
      pltpu.VMEM_SHARED((spad,), jnp.float32),
      pltpu.VMEM_SHARED((spad,), jnp.float32),
      pltpu.VMEM_SHARED((cw + 16, D), jnp.float32),
      pltpu.SemaphoreType.DMA,
  ]
  if linear_fin:
    scratch += [
        pltpu.VMEM((_FCH, D), jnp.float32),
        pltpu.VMEM((_FCH, D), jnp.float32),
    ]
  else:
    scratch += [
        pltpu.VMEM((_FCAP,), jnp.int32),
        pltpu.VMEM((_FCAP,), jnp.int32),
        pltpu.VMEM((_FCAP,), jnp.float32),
        pltpu.VMEM((16, D), jnp.float32),
        pltpu.VMEM((16, D), jnp.float32),
    ]

  def body(out_ref, ids_hbm, e_hbm, ev_hbm, bemb_hbm,
           dummy_hbm, hidl_hbm, hocc_hbm, *scr):
    if linear_fin:
      (ids_v, e_v, idx_v, ones_v, zeros_v, cg_v, g1_v, b1_v, hil, hoc,
       wpos, wocc, rows_v, posb, gi16, pi16, cnb, dnb,
       cnt_sh, den_sh, ev_sh, sem, evb, qb) = scr
    else:
      (ids_v, e_v, idx_v, ones_v, zeros_v, cg_v, g1_v, b1_v, hil, hoc,
       wpos, wocc, rows_v, posb, gi16, pi16, cnb, dnb,
       cnt_sh, den_sh, ev_sh, sem, gq_v, pw_v, dv_v, qrow, evrow) = scr

    cid = lax.axis_index("c")
    sid = lax.axis_index("s")
    wid = cid * NS + sid
    base = cid * half
    iot = lax.iota(jnp.int32, L)

    def fill(j, _):
      ones_v[pl.ds(j * L, L)] = jnp.full((L,), 1.0, jnp.float32)
      zeros_v[pl.ds(j * L, L)] = jnp.zeros((L,), jnp.float32)
      return 0
    lax.fori_loop(0, _CH // L, fill, 0)

    zpt = spad // NS
    def zc(c, _):
      o = sid * zpt + c * _CH
      pltpu.sync_copy(zeros_v, cnt_sh.at[pl.ds(o, _CH)])
      pltpu.sync_copy(zeros_v, den_sh.at[pl.ds(o, _CH)])
      return 0
    lax.fori_loop(0, zpt // _CH, zc, 0)
    plsc.subcore_barrier()

    def c1(c, _):
      off = sid * _SOCC + c * _CH
      pltpu.sync_copy(ids_hbm.at[pl.ds(off, _CH)], ids_v)
      pltpu.sync_copy(e_hbm.at[pl.ds(off, _CH)], e_v)
      def vl(j, _):
        iv = ids_v[pl.ds(j * L, L)]
        m = (iv >= base) & (iv < base + half)
        idx_v[pl.ds(j * L, L)] = jnp.where(m, iv - base, span)
        return 0
      lax.fori_loop(0, _CH // L, vl, 0)
      pltpu.sync_copy(ones_v, cnt_sh.at[idx_v], add=True)
      pltpu.sync_copy(e_v, den_sh.at[idx_v], add=True)
      return 0
    lax.fori_loop(0, _NCHUNK, c1, 0)
    plsc.subcore_barrier()

    def c2(c, carry):
      staged, done = carry
      off = sid * _SOCC + c * _CH
      pltpu.sync_copy(ids_hbm.at[pl.ds(off, _CH)], ids_v)
      def vl(j, _):
        iv = ids_v[pl.ds(j * L, L)]
        m = (iv >= base) & (iv < base + half)
        idx_v[pl.ds(j * L, L)] = jnp.where(m, iv - base, span)
        return 0
      lax.fori_loop(0, _CH // L, vl, 0)
      pltpu.async_copy(cnt_sh.at[idx_v], cg_v, sem).wait()

      def vl2(j, carry2):
        cur1, st = carry2
        iv = ids_v[pl.ds(j * L, L)]
        idl = idx_v[pl.ds(j * L, L)]
        cg = cg_v[pl.ds(j * L, L)]
        m_in = idl < span
        occ = off + j * L + iot
        bas = occ // BK
        m1 = m_in & (cg == 1.0)
        cur1 = _compress(m1, cur1, _G1TRASH, [(g1_v, iv), (b1_v, bas)])
        m2 = m_in & (cg >= 2.0)
        st = _compress(m2, st, _HST - 16, [(hil, idl), (hoc, occ)])
        return (cur1, st)
      n1, staged = lax.fori_loop(0, _CH // L, vl2, (jnp.int32(0), staged))

      @pl.when(staged >= _Q)
      def _():
        dd = pl.multiple_of(done, _Q)
        pltpu.sync_copy(hil.at[pl.ds(0, _Q)], hidl_hbm.at[wid, pl.ds(dd, _Q)])
        pltpu.sync_copy(hoc.at[pl.ds(0, _Q)], hocc_hbm.at[wid, pl.ds(dd, _Q)])
        rem = staged - _Q
        def mv(k, _):
          hil[pl.ds(k * L, L)] = hil[pl.ds(_Q + k * L, L)]
          hoc[pl.ds(k * L, L)] = hoc[pl.ds(_Q + k * L, L)]
          return 0
        lax.fori_loop(0, (rem + L - 1) // L, mv, 0)
      spill = (staged >= _Q).astype(jnp.int32)
      staged = staged - spill * _Q
      done = done + spill * _Q

      @pl.when(n1 > 0)
      def _():
        g0 = g1_v[pl.ds(0, L)][0]
        b0 = b1_v[pl.ds(0, L)][0]
        def pad(t, _):
          pos = n1 + t * L + iot
          plsc.store_scatter(g1_v, [pos], jnp.full((L,), g0, jnp.int32))
          plsc.store_scatter(b1_v, [pos], jnp.full((L,), b0, jnp.int32))
          return 0
        lax.fori_loop(0, _PB // L, pad, 0)
        def piece(p, _):
          def cpb(k, _):
            posb[pl.ds(k * L, L)] = b1_v[pl.ds(p * _PB + k * L, L)]
            return 0
          lax.fori_loop(0, _PB // L, cpb, 0)
          pltpu.async_copy(bemb_hbm.at[posb], rows_v, sem).wait()
          def cp(k, _):
            posb[pl.ds(k * L, L)] = g1_v[pl.ds(p * _PB + k * L, L)]
            return 0
          lax.fori_loop(0, _PB // L, cp, 0)
          pltpu.async_copy(rows_v, out_ref.at[posb], sem).wait()
          return 0
        lax.fori_loop(0, (n1 + _PB - 1) // _PB, piece, 0)
      return (staged, done)

    staged, done = lax.fori_loop(0, _NCHUNK, c2,
                                 (jnp.int32(0), jnp.int32(0)))

    def tpad(t, _):
      pos = staged + t * L + iot
      pos = jnp.where(pos < _Q, pos, _HST - 16)
      plsc.store_scatter(hil, [pos], jnp.full((L,), span, jnp.int32))
      plsc.store_scatter(hoc, [pos], jnp.zeros((L,), jnp.int32))
      return 0
    lax.fori_loop(0, (_Q - staged + L - 1) // L, tpad, 0)
    ddf = pl.multiple_of(done, _Q)
    pltpu.sync_copy(hil.at[pl.ds(0, _Q)], hidl_hbm.at[wid, pl.ds(ddf, _Q)])
    pltpu.sync_copy(hoc.at[pl.ds(0, _Q)], hocc_hbm.at[wid, pl.ds(ddf, _Q)])
    nblk = done // _Q + 1

    def zero_rows_buf(buf, n):
      def zr(r, _):
        row = buf.at[r]
        row[pl.ds(0, L)] = jnp.zeros((L,), jnp.float32)
        row[pl.ds(L, L)] = jnp.zeros((L,), jnp.float32)
        return 0
      lax.fori_loop(0, n, zr, 0)

    def window(w, _):
      w0 = w * cw

      def scan_blocks(emit, need_occ):
        def blk(b, _):
          bo = pl.multiple_of(b * _Q, _Q)
          pltpu.sync_copy(hidl_hbm.at[wid, pl.ds(bo, _Q)],
                          hil.at[pl.ds(0, _Q)])
          if need_occ:
            pltpu.sync_copy(hocc_hbm.at[wid, pl.ds(bo, _Q)],
                            hoc.at[pl.ds(0, _Q)])
          def hs(t, wn):
            hv = hil[pl.ds(t * L, L)]
            ov = hoc[pl.ds(t * L, L)]
            m = (hv >= w0) & (hv < w0 + cw)
            return _compress(m, wn, _WTRASH,
                             [(wpos, hv - w0), (wocc, ov)])
          wn = lax.fori_loop(0, _Q // L, hs, jnp.int32(0))
          def wpad(t, _):
            pos = wn + t * L + iot
            plsc.store_scatter(wpos, [pos], jnp.full((L,), cw, jnp.int32))
            plsc.store_scatter(wocc, [pos], jnp.zeros((L,), jnp.int32))
            return 0
          lax.fori_loop(0, _PB // L, wpad, 0)
          def pc(q, _):
            emit(q)
            return 0
          lax.fori_loop(0, (wn + _PB - 1) // _PB, pc, 0)
          return 0
        lax.fori_loop(0, nblk, blk, 0)

      def cp_posb(src, q):
        def cp(k, _):
          posb[pl.ds(k * L, L)] = src[pl.ds(q * _PB + k * L, L)]
          return 0
        lax.fori_loop(0, _PB // L, cp, 0)

      if linear_fin:
        zero_rows_buf(evb, _FCH)
        def zl(c, _):
          pltpu.sync_copy(evb, ev_sh.at[pl.ds(sid * fw + c * _FCH, _FCH)])
          return 0
        lax.fori_loop(0, fw // _FCH, zl, 0)
      else:
        zero_rows_buf(rows_v, _PB)
        def zemit(q):
          cp_posb(wpos, q)
          pltpu.sync_copy(rows_v, ev_sh.at[posb])
        scan_blocks(zemit, need_occ=False)
      plsc.subcore_barrier()

      def aemit(q):
        cp_posb(wocc, q)
        pltpu.async_copy(ev_hbm.at[posb], rows_v, sem).wait()
        cp_posb(wpos, q)
        pltpu.sync_copy(rows_v, ev_sh.at[posb], add=True)
      scan_blocks(aemit, need_occ=True)
      plsc.subcore_barrier()

      def fin(c, _):
        lo = w0 + sid * fw + c * _FCH
        @pl.when(lo < half)
        def _():
          pltpu.sync_copy(cnt_sh.at[pl.ds(lo, _FCH)], cnb.at[pl.ds(0, _FCH)])
          pltpu.sync_copy(den_sh.at[pl.ds(lo, _FCH)], dnb.at[pl.ds(0, _FCH)])
          if linear_fin:
            pltpu.sync_copy(ev_sh.at[pl.ds(lo - w0, _FCH)], evb)
            pltpu.sync_copy(out_ref.at[pl.ds(base + lo, _FCH)], qb)
            def frow(j, _):
              cnvec = cnb[pl.ds(j * L, L)]
              recvec = 1.0 / dnb[pl.ds(j * L, L)]
              for r in range(L):
                rec = recvec[r]
                mv = jnp.full((L,), cnvec[r]) >= 2.0
                qr = qb.at[j * L + r]
                er = evb.at[j * L + r]
                for h in (0, L):
                  val = qr[pl.ds(h, L)] + er[pl.ds(h, L)] * rec
                  qr[pl.ds(h, L)] = jnp.where(mv, val, qr[pl.ds(h, L)])
              return 0
            lax.fori_loop(0, _FCH // L, frow, 0)
            pltpu.sync_copy(qb, out_ref.at[pl.ds(base + lo, _FCH)])
          else:
            def fscan(j, nf):
              cg = cnb[pl.ds(j * L, L)]
              dv = dnb[pl.ds(j * L, L)]
              m = cg >= 2.0
              gid = base + lo + j * L + iot
              pw = (lo - w0) + j * L + iot
              return _compress(m, nf, _FTRASH,
                               [(gq_v, gid), (pw_v, pw), (dv_v, dv)])
            nf = lax.fori_loop(0, _FCH // L, fscan, jnp.int32(0))
            @pl.when(nf > 0)
            def _():
              g0 = gq_v[pl.ds(0, L)][0]
              p0 = pw_v[pl.ds(0, L)][0]
              d0 = dv_v[pl.ds(0, L)][0]
              pos = nf + iot
              plsc.store_scatter(gq_v, [pos], jnp.full((L,), g0, jnp.int32))
              plsc.store_scatter(pw_v, [pos], jnp.full((L,), p0, jnp.int32))
              plsc.store_scatter(dv_v, [pos], jnp.full((L,), d0, jnp.float32))
              def fp(p, _):
                gi16[pl.ds(0, L)] = gq_v[pl.ds(p * L, L)]
                pi16[pl.ds(0, L)] = pw_v[pl.ds(p * L, L)]
                pltpu.async_copy(out_ref.at[gi16], qrow, sem).wait()
                pltpu.async_copy(ev_sh.at[pi16], evrow, sem).wait()
                recvec = 1.0 / dv_v[pl.ds(p * L, L)]
                for r in range(L):
                  rec = recvec[r]
                  qr = qrow.at[r]
                  er = evrow.at[r]
                  for h in (0, L):
                    qr[pl.ds(h, L)] = qr[pl.ds(h, L)] + er[pl.ds(h, L)] * rec
                pltpu.async_copy(qrow, out_ref.at[gi16], sem).wait()
                return 0
              lax.fori_loop(0, (nf + L - 1) // L, fp, 0)
        return 0
      lax.fori_loop(0, fw // _FCH, fin, 0)
      plsc.subcore_barrier()
      return 0

    lax.fori_loop(0, windows, window, 0)

    @pl.when((cid == 0) & (sid == 0))
    def _():
      pltpu.sync_copy(zeros_v.at[pl.ds(0, L)], dummy_hbm)

  return functools.partial(
      pl.kernel, mesh=_MESH, compiler_params=_CP,
      out_type=(
          jax.ShapeDtypeStruct((L,), jnp.float32),
          jax.ShapeDtypeStruct((NC * NS, _SOCC + _Q), jnp.int32),
          jax.ShapeDtypeStruct((NC * NS, _SOCC + _Q), jnp.int32),
      ),
      scratch_types=scratch,
  )(body)


_c_items = _make_phase_c(N_IT, cw=12_800, windows=40, spad=_SPAD_I,
                         linear_fin=False)
_c_prices = _make_phase_c(N_PR, cw=25_600, windows=2, spad=_SPAD_P,
                          linear_fin=True)


def kernel(item_embeddings, price_embeddings, category_embeddings, samples,
           sampleLen, Wk_item, Wv_item, Wk_price, Wv_price):
  del category_embeddings, sampleLen
  sf = samples.reshape(-1).astype(jnp.int32)

  qi, qp, pids = _phase_a(sf, item_embeddings, price_embeddings)

  bi, ei, evi, bp, ep, evp = _phase_b(
      qi.reshape(NB, BK, D), qp.reshape(NB, BK, D),
      Wk_item, Wv_item, Wk_price, Wv_price)

  ref_i = jax.new_ref(item_embeddings)
  ref_p = jax.new_ref(price_embeddings)
  _c_items(ref_i, sf, ei.reshape(-1), evi.reshape(NOCC, D), bi)
  _c_prices(ref_p, pids, ep.reshape(-1), evp.reshape(NOCC, D), bp)
  new_item = ref_i[...]
  new_price = ref_p[...]

  return (new_item, new_price)


# speedup vs baseline: 2.9684x; 1.0340x over previous
import functools
import math

import jax
import jax.numpy as jnp
from jax import lax
from jax.experimental import pallas as pl
from jax.experimental.pallas import tpu as pltpu, tpu_sc as plsc

NC, NS, L = 2, 16, 16
NB, BK, D = 16384, 20, 32
NOCC = NB * BK
N_IT, N_PR = 1_000_000, 100_000

_MESH = plsc.VectorSubcoreMesh(core_axis_name="c", subcore_axis_name="s",
                               num_cores=NC, num_subcores=NS)
_CP = pltpu.CompilerParams(use_tc_tiling_on_sc=False, needs_layout_passes=False)


_OPT = NOCC // (NC * NS)
_CHA = 2048


@functools.partial(
    pl.kernel, mesh=_MESH, compiler_params=_CP,
    out_type=(
        jax.ShapeDtypeStruct((NOCC, D), jnp.float32),
        jax.ShapeDtypeStruct((NOCC, D), jnp.float32),
        jax.ShapeDtypeStruct((NOCC,), jnp.int32),
    ),
    scratch_types=[
        pltpu.VMEM((_CHA,), jnp.int32),
        pltpu.VMEM((_CHA,), jnp.int32),
        pltpu.VMEM((_CHA, D), jnp.float32),
        pltpu.SemaphoreType.DMA,
    ],
)
def _phase_a(sf_hbm, item_hbm, price_hbm, qi_hbm, qp_hbm, pid_hbm,
             ids_v, pids_v, rows_v, sem):
  cid = lax.axis_index("c")
  sid = lax.axis_index("s")
  wid = cid * NS + sid

  def chunk(c, _):
    off = wid * _OPT + c * _CHA
    pltpu.sync_copy(sf_hbm.at[pl.ds(off, _CHA)], ids_v)
    pltpu.async_copy(item_hbm.at[ids_v], rows_v, sem).wait()
    pltpu.sync_copy(rows_v, qi_hbm.at[pl.ds(off, _CHA)])

    def pv(j, _):
      iv = ids_v[pl.ds(j * L, L)]
      pids_v[pl.ds(j * L, L)] = iv % N_PR
      return 0
    lax.fori_loop(0, _CHA // L, pv, 0)
    pltpu.sync_copy(pids_v, pid_hbm.at[pl.ds(off, _CHA)])
    pltpu.async_copy(price_hbm.at[pids_v], rows_v, sem).wait()
    pltpu.sync_copy(rows_v, qp_hbm.at[pl.ds(off, _CHA)])
    return 0

  lax.fori_loop(0, _OPT // _CHA, chunk, 0)


_BB = 512
_ISQ = 1.0 / math.sqrt(D)


def _phase_b_body(qi_ref, qp_ref, wki_ref, wvi_ref, wkp_ref, wvp_ref,
                  bi_ref, ei_ref, evi_ref, bp_ref, ep_ref, evp_ref):
  def tab(qref, wkref, wvref, bref, eref, evref):
    q = qref[...]
    s = jnp.sum(q, axis=1) * (1.0 / BK)
    b = jnp.tanh(s)
    k = jnp.dot(b, wkref[...], preferred_element_type=jnp.float32)
    v = jnp.dot(b, wvref[...], preferred_element_type=jnp.float32)
    sc = jnp.sum(q * k[:, None, :], axis=-1) * _ISQ
    e = jnp.exp(sc)
    bref[...] = b
    eref[...] = e
    evref[...] = e[:, :, None] * v[:, None, :]

  tab(qi_ref, wki_ref, wvi_ref, bi_ref, ei_ref, evi_ref)
  tab(qp_ref, wkp_ref, wvp_ref, bp_ref, ep_ref, evp_ref)


def _phase_b(qi3, qp3, wki, wvi, wkp, wvp):
  wspec = pl.BlockSpec((D, D), lambda i: (0, 0))
  return pl.pallas_call(
      _phase_b_body,
      grid=(NB // _BB,),
      in_specs=[
          pl.BlockSpec((_BB, BK, D), lambda i: (i, 0, 0)),
          pl.BlockSpec((_BB, BK, D), lambda i: (i, 0, 0)),
          wspec, wspec, wspec, wspec,
      ],
      out_specs=[
          pl.BlockSpec((_BB, D), lambda i: (i, 0)),
          pl.BlockSpec((_BB, BK), lambda i: (i, 0)),
          pl.BlockSpec((_BB, BK, D), lambda i: (i, 0, 0)),
          pl.BlockSpec((_BB, D), lambda i: (i, 0)),
          pl.BlockSpec((_BB, BK), lambda i: (i, 0)),
          pl.BlockSpec((_BB, BK, D), lambda i: (i, 0, 0)),
      ],
      out_shape=[
          jax.ShapeDtypeStruct((NB, D), jnp.float32),
          jax.ShapeDtypeStruct((NB, BK), jnp.float32),
          jax.ShapeDtypeStruct((NB, BK, D), jnp.float32),
          jax.ShapeDtypeStruct((NB, D), jnp.float32),
          jax.ShapeDtypeStruct((NB, BK), jnp.float32),
          jax.ShapeDtypeStruct((NB, BK, D), jnp.float32),
      ],
  )(qi3, qp3, wki, wvi, wkp, wvp)


_SOCC = NOCC // NS
_CH = 1024
_NCHUNK = _SOCC // _CH
_PB = 256
_FCH = 400
_Q = 4096
_HST = _Q + _CH + 16
_WCAP = _Q + _PB + 16
_WTRASH = _Q + _PB
_G1CAP = _CH + _PB + 16
_G1TRASH = _CH + _PB
_FCAP = _FCH + 16 + 16
_FTRASH = _FCH + 16
_SPAD_I, _SPAD_P = 524_288, 65_536


def _compress(m, cursor, trash, pairs):
  mi = m.astype(jnp.int32)
  c = plsc.cumsum(mi)
  n = jnp.sum(mi)
  pos = jnp.where(m, cursor + c - 1, trash)
  for ref, vec in pairs:
    plsc.store_scatter(ref, [pos], vec)
  return cursor + n


def _make_phase_c(n_rows, cw, windows, spad, linear_fin):
  half = n_rows // 2
  span = windows * cw
  fw = cw // NS

  scratch = [
      pltpu.VMEM((_CH,), jnp.int32),
      pltpu.VMEM((_CH,), jnp.float32),
      pltpu.VMEM((_CH,), jnp.int32),
      pltpu.VMEM((_CH,), jnp.float32),
      pltpu.VMEM((_CH,), jnp.float32),
      pltpu.VMEM((_CH,), jnp.float32),
      pltpu.VMEM((_G1CAP,), jnp.int32),
      pltpu.VMEM((_G1CAP,), jnp.int32),
      pltpu.VMEM((_HST,), jnp.int32),
      pltpu.VMEM((_HST,), jnp.int32),
      pltpu.VMEM((_WCAP,), jnp.int32),
      pltpu.VMEM((_WCAP,), jnp.int32),
      pltpu.VMEM((_PB, D), jnp.float32),
      pltpu.VMEM((_PB,), jnp.int32),
      pltpu.VMEM((16,), jnp.int32),
      pltpu.VMEM((16,), jnp.int32),
      pltpu.VMEM((_FCH + 16,), jnp.float32),
      pltpu.VMEM((_FCH + 16,), jnp.float32),
      pltpu.VMEM_SHARED((spad,), jnp.float32),
      pltpu.VMEM_SHARED((spad,), jnp.float32),
      pltpu.VMEM_SHARED((cw + 16, D), jnp.float32),
      pltpu.SemaphoreType.DMA,
  ]
  if linear_fin:
    scratch += [
        pltpu.VMEM((_FCH, D), jnp.float32),
        pltpu.VMEM((_FCH, D), jnp.float32),
    ]
  else:
    scratch += [
        pltpu.VMEM((_FCAP,), jnp.int32),
        pltpu.VMEM((_FCAP,), jnp.int32),
        pltpu.VMEM((_FCAP,), jnp.float32),
        pltpu.VMEM((16, D), jnp.float32),
        pltpu.VMEM((16, D), jnp.float32),
    ]

  def body(out_ref, ids_hbm, e_hbm, ev_hbm, bemb_hbm,
           dummy_hbm, hidl_hbm, hocc_hbm, *scr):
    if linear_fin:
      (ids_v, e_v, idx_v, ones_v, zeros_v, cg_v, g1_v, b1_v, hil, hoc,
       wpos, wocc, rows_v, posb, gi16, pi16, cnb, dnb,
       cnt_sh, den_sh, ev_sh, sem, evb, qb) = scr
    else:
      (ids_v, e_v, idx_v, ones_v, zeros_v, cg_v, g1_v, b1_v, hil, hoc,
       wpos, wocc, rows_v, posb, gi16, pi16, cnb, dnb,
       cnt_sh, den_sh, ev_sh, sem, gq_v, pw_v, dv_v, qrow, evrow) = scr

    cid = lax.axis_index("c")
    sid = lax.axis_index("s")
    wid = cid * NS + sid
    base = cid * half
    iot = lax.iota(jnp.int32, L)

    def fill(j, _):
      ones_v[pl.ds(j * L, L)] = jnp.full((L,), 1.0, jnp.float32)
      zeros_v[pl.ds(j * L, L)] = jnp.zeros((L,), jnp.float32)
      return 0
    lax.fori_loop(0, _CH // L, fill, 0)

    zpt = spad // NS
    def zc(c, _):
      o = sid * zpt + c * _CH
      pltpu.sync_copy(zeros_v, cnt_sh.at[pl.ds(o, _CH)])
      pltpu.sync_copy(zeros_v, den_sh.at[pl.ds(o, _CH)])
      return 0
    lax.fori_loop(0, zpt // _CH, zc, 0)
    plsc.subcore_barrier()

    def c1(c, _):
      off = sid * _SOCC + c * _CH
      pltpu.sync_copy(ids_hbm.at[pl.ds(off, _CH)], ids_v)
      pltpu.sync_copy(e_hbm.at[pl.ds(off, _CH)], e_v)
      def vl(j, _):
        iv = ids_v[pl.ds(j * L, L)]
        m = (iv >= base) & (iv < base + half)
        idx_v[pl.ds(j * L, L)] = jnp.where(m, iv - base, span)
        return 0
      lax.fori_loop(0, _CH // L, vl, 0)
      pltpu.sync_copy(ones_v, cnt_sh.at[idx_v], add=True)
      pltpu.sync_copy(e_v, den_sh.at[idx_v], add=True)
      return 0
    lax.fori_loop(0, _NCHUNK, c1, 0)
    plsc.subcore_barrier()

    def c2(c, carry):
      staged, done = carry
      off = sid * _SOCC + c * _CH
      pltpu.sync_copy(ids_hbm.at[pl.ds(off, _CH)], ids_v)
      def vl(j, _):
        iv = ids_v[pl.ds(j * L, L)]
        m = (iv >= base) & (iv < base + half)
        idx_v[pl.ds(j * L, L)] = jnp.where(m, iv - base, span)
        return 0
      lax.fori_loop(0, _CH // L, vl, 0)
      pltpu.async_copy(cnt_sh.at[idx_v], cg_v, sem).wait()

      def vl2(j, carry2):
        cur1, st = carry2
        iv = ids_v[pl.ds(j * L, L)]
        idl = idx_v[pl.ds(j * L, L)]
        cg = cg_v[pl.ds(j * L, L)]
        m_in = idl < span
        occ = off + j * L + iot
        bas = occ // BK
        m1 = m_in & (cg == 1.0)
        cur1 = _compress(m1, cur1, _G1TRASH, [(g1_v, iv), (b1_v, bas)])
        m2 = m_in & (cg >= 2.0)
        st = _compress(m2, st, _HST - 16, [(hil, idl), (hoc, occ)])
        return (cur1, st)
      n1, staged = lax.fori_loop(0, _CH // L, vl2, (jnp.int32(0), staged))

      @pl.when(staged >= _Q)
      def _():
        dd = pl.multiple_of(done, _Q)
        pltpu.sync_copy(hil.at[pl.ds(0, _Q)], hidl_hbm.at[wid, pl.ds(dd, _Q)])
        pltpu.sync_copy(hoc.at[pl.ds(0, _Q)], hocc_hbm.at[wid, pl.ds(dd, _Q)])
        rem = staged - _Q
        def mv(k, _):
          hil[pl.ds(k * L, L)] = hil[pl.ds(_Q + k * L, L)]
          hoc[pl.ds(k * L, L)] = hoc[pl.ds(_Q + k * L, L)]
          return 0
        lax.fori_loop(0, (rem + L - 1) // L, mv, 0)
      spill = (staged >= _Q).astype(jnp.int32)
      staged = staged - spill * _Q
      done = done + spill * _Q

      @pl.when(n1 > 0)
      def _():
        g0 = g1_v[pl.ds(0, L)][0]
        b0 = b1_v[pl.ds(0, L)][0]
        def pad(t, _):
          pos = n1 + t * L + iot
          plsc.store_scatter(g1_v, [pos], jnp.full((L,), g0, jnp.int32))
          plsc.store_scatter(b1_v, [pos], jnp.full((L,), b0, jnp.int32))
          return 0
        lax.fori_loop(0, _PB // L, pad, 0)
        def piece(p, _):
          def cpb(k, _):
            posb[pl.ds(k * L, L)] = b1_v[pl.ds(p * _PB + k * L, L)]
            return 0
          lax.fori_loop(0, _PB // L, cpb, 0)
          pltpu.async_copy(bemb_hbm.at[posb], rows_v, sem).wait()
          def cp(k, _):
            posb[pl.ds(k * L, L)] = g1_v[pl.ds(p * _PB + k * L, L)]
            return 0
          lax.fori_loop(0, _PB // L, cp, 0)
          pltpu.async_copy(rows_v, out_ref.at[posb], sem).wait()
          return 0
        lax.fori_loop(0, (n1 + _PB - 1) // _PB, piece, 0)
      return (staged, done)

    staged, done = lax.fori_loop(0, _NCHUNK, c2,
                                 (jnp.int32(0), jnp.int32(0)))

    def tpad(t, _):
      pos = staged + t * L + iot
      pos = jnp.where(pos < _Q, pos, _HST - 16)
      plsc.store_scatter(hil, [pos], jnp.full((L,), span, jnp.int32))
      plsc.store_scatter(hoc, [pos], jnp.zeros((L,), jnp.int32))
      return 0
    lax.fori_loop(0, (_Q - staged + L - 1) // L, tpad, 0)
    ddf = pl.multiple_of(done, _Q)
    pltpu.sync_copy(hil.at[pl.ds(0, _Q)], hidl_hbm.at[wid, pl.ds(ddf, _Q)])
    pltpu.sync_copy(hoc.at[pl.ds(0, _Q)], hocc_hbm.at[wid, pl.ds(ddf, _Q)])
    nblk = done // _Q + 1

    def zero_rows_buf(buf, n):
      def zr(r, _):
        row = buf.at[r]
        row[pl.ds(0, L)] = jnp.zeros((L,), jnp.float32)
        row[pl.ds(L, L)] = jnp.zeros((L,), jnp.float32)
        return 0
      lax.fori_loop(0, n, zr, 0)

    def window(w, _):
      w0 = w * cw

      def scan_blocks(emit):
        def blk(b, _):
          bo = pl.multiple_of(b * _Q, _Q)
          pltpu.sync_copy(hidl_hbm.at[wid, pl.ds(bo, _Q)],
                          hil.at[pl.ds(0, _Q)])
          pltpu.sync_copy(hocc_hbm.at[wid, pl.ds(bo, _Q)],
                          hoc.at[pl.ds(0, _Q)])
          def hs(t, wn):
            hv = hil[pl.ds(t * L, L)]
            ov = hoc[pl.ds(t * L, L)]
            m = (hv >= w0) & (hv < w0 + cw)
            return _compress(m, wn, _WTRASH,
                             [(wpos, hv - w0), (wocc, ov)])
          wn = lax.fori_loop(0, _Q // L, hs, jnp.int32(0))
          def wpad(t, _):
            pos = wn + t * L + iot
            plsc.store_scatter(wpos, [pos], jnp.full((L,), cw, jnp.int32))
            plsc.store_scatter(wocc, [pos], jnp.zeros((L,), jnp.int32))
            return 0
          lax.fori_loop(0, _PB // L, wpad, 0)
          def pc(q, _):
            emit(q)
            return 0
          lax.fori_loop(0, (wn + _PB - 1) // _PB, pc, 0)
          return 0
        lax.fori_loop(0, nblk, blk, 0)

      def cp_posb(src, q):
        def cp(k, _):
          posb[pl.ds(k * L, L)] = src[pl.ds(q * _PB + k * L, L)]
          return 0
        lax.fori_loop(0, _PB // L, cp, 0)

      if linear_fin:
        zero_rows_buf(evb, _FCH)
        def zl(c, _):
          pltpu.sync_copy(evb, ev_sh.at[pl.ds(sid * fw + c * _FCH, _FCH)])
          return 0
        lax.fori_loop(0, fw // _FCH, zl, 0)
      else:
        zero_rows_buf(rows_v, _PB)
        zo = 0
        while zo < fw:
          zs = min(_PB, fw - zo)
          pltpu.sync_copy(rows_v.at[pl.ds(0, zs)],
                          ev_sh.at[pl.ds(sid * fw + zo, zs)])
          zo += zs
      plsc.subcore_barrier()

      def aemit(q):
        cp_posb(wocc, q)
        pltpu.async_copy(ev_hbm.at[posb], rows_v, sem).wait()
        cp_posb(wpos, q)
        pltpu.sync_copy(rows_v, ev_sh.at[posb], add=True)
      scan_blocks(aemit)
      plsc.subcore_barrier()

      def fin(c, _):
        lo = w0 + sid * fw + c * _FCH
        @pl.when(lo < half)
        def _():
          pltpu.sync_copy(cnt_sh.at[pl.ds(lo, _FCH)], cnb.at[pl.ds(0, _FCH)])
          pltpu.sync_copy(den_sh.at[pl.ds(lo, _FCH)], dnb.at[pl.ds(0, _FCH)])
          if linear_fin:
            pltpu.sync_copy(ev_sh.at[pl.ds(lo - w0, _FCH)], evb)
            pltpu.sync_copy(out_ref.at[pl.ds(base + lo, _FCH)], qb)
            def frow(j, _):
              cnvec = cnb[pl.ds(j * L, L)]
              recvec = 1.0 / dnb[pl.ds(j * L, L)]
              for r in range(L):
                rec = recvec[r]
                mv = jnp.full((L,), cnvec[r]) >= 2.0
                qr = qb.at[j * L + r]
                er = evb.at[j * L + r]
                for h in (0, L):
                  val = qr[pl.ds(h, L)] + er[pl.ds(h, L)] * rec
                  qr[pl.ds(h, L)] = jnp.where(mv, val, qr[pl.ds(h, L)])
              return 0
            lax.fori_loop(0, _FCH // L, frow, 0)
            pltpu.sync_copy(qb, out_ref.at[pl.ds(base + lo, _FCH)])
          else:
            def fscan(j, nf):
              cg = cnb[pl.ds(j * L, L)]
              dv = dnb[pl.ds(j * L, L)]
              m = cg >= 2.0
              gid = base + lo + j * L + iot
              pw = (lo - w0) + j * L + iot
              return _compress(m, nf, _FTRASH,
                               [(gq_v, gid), (pw_v, pw), (dv_v, dv)])
            nf = lax.fori_loop(0, _FCH // L, fscan, jnp.int32(0))
            @pl.when(nf > 0)
            def _():
              g0 = gq_v[pl.ds(0, L)][0]
              p0 = pw_v[pl.ds(0, L)][0]
              d0 = dv_v[pl.ds(0, L)][0]
              pos = nf + iot
              plsc.store_scatter(gq_v, [pos], jnp.full((L,), g0, jnp.int32))
              plsc.store_scatter(pw_v, [pos], jnp.full((L,), p0, jnp.int32))
              plsc.store_scatter(dv_v, [pos], jnp.full((L,), d0, jnp.float32))
              def fp(p, _):
                gi16[pl.ds(0, L)] = gq_v[pl.ds(p * L, L)]
                pi16[pl.ds(0, L)] = pw_v[pl.ds(p * L, L)]
                pltpu.async_copy(out_ref.at[gi16], qrow, sem).wait()
                pltpu.async_copy(ev_sh.at[pi16], evrow, sem).wait()
                recvec = 1.0 / dv_v[pl.ds(p * L, L)]
                for r in range(L):
                  rec = recvec[r]
                  qr = qrow.at[r]
                  er = evrow.at[r]
                  for h in (0, L):
                    qr[pl.ds(h, L)] = qr[pl.ds(h, L)] + er[pl.ds(h, L)] * rec
                pltpu.async_copy(qrow, out_ref.at[gi16], sem).wait()
                return 0
              lax.fori_loop(0, (nf + L - 1) // L, fp, 0)
        return 0
      lax.fori_loop(0, fw // _FCH, fin, 0)
      plsc.subcore_barrier()
      return 0

    lax.fori_loop(0, windows, window, 0)

    @pl.when((cid == 0) & (sid == 0))
    def _():
      pltpu.sync_copy(zeros_v.at[pl.ds(0, L)], dummy_hbm)

  return functools.partial(
      pl.kernel, mesh=_MESH, compiler_params=_CP,
      out_type=(
          jax.ShapeDtypeStruct((L,), jnp.float32),
          jax.ShapeDtypeStruct((NC * NS, _SOCC + _Q), jnp.int32),
          jax.ShapeDtypeStruct((NC * NS, _SOCC + _Q), jnp.int32),
      ),
      scratch_types=scratch,
  )(body)


_c_items = _make_phase_c(N_IT, cw=12_800, windows=40, spad=_SPAD_I,
                         linear_fin=False)
_c_prices = _make_phase_c(N_PR, cw=25_600, windows=2, spad=_SPAD_P,
                          linear_fin=True)


def kernel(item_embeddings, price_embeddings, category_embeddings, samples,
           sampleLen, Wk_item, Wv_item, Wk_price, Wv_price):
  del category_embeddings, sampleLen
  sf = samples.reshape(-1).astype(jnp.int32)

  qi, qp, pids = _phase_a(sf, item_embeddings, price_embeddings)

  bi, ei, evi, bp, ep, evp = _phase_b(
      qi.reshape(NB, BK, D), qp.reshape(NB, BK, D),
      Wk_item, Wv_item, Wk_price, Wv_price)

  ref_i = jax.new_ref(item_embeddings)
  ref_p = jax.new_ref(price_embeddings)
  _c_items(ref_i, sf, ei.reshape(-1), evi.reshape(NOCC, D), bi)
  _c_prices(ref_p, pids, ep.reshape(-1), evp.reshape(NOCC, D), bp)
  new_item = ref_i[...]
  new_price = ref_p[...]

  return (new_item, new_price)
